# baseline (speedup 1.0000x reference)
import functools

import jax
import jax.numpy as jnp
from jax.experimental import pallas as pl
from jax.experimental.pallas import tpu as pltpu

F = 128
C = 128
BM = 320


def _fused_body(fltr_ref, feat_ref, wa_ref, ba_ref, wb_ref, bb_ref,
                awa_ref, awb_ref, out_ref):
    h = jnp.dot(fltr_ref[...], feat_ref[...],
                preferred_element_type=jnp.float32)
    a = jnp.maximum(
        jnp.dot(h, wa_ref[...], preferred_element_type=jnp.float32)
        + ba_ref[...], 0.0)
    b = jnp.maximum(
        jnp.dot(h, wb_ref[...], preferred_element_type=jnp.float32)
        + bb_ref[...], 0.0)
    mx = jnp.maximum(a, b)
    av = 0.5 * (a + b)
    gate = jax.nn.sigmoid(
        jnp.sum(a * awa_ref[...] + b * awb_ref[...], axis=1, keepdims=True))
    pooled = gate * mx + (1.0 - gate) * av
    y = jnp.where(pooled > 0, pooled, jnp.exp(pooled) - 1.0) + pooled
    out_ref[...] = jnp.where(y > 0, y, jnp.exp(y) - 1.0)


@functools.partial(jax.jit, static_argnames=())
def kernel(features, fltr, W0, b0, W1, b1, alpha_w):
    n = fltr.shape[0]
    wa = jnp.concatenate([W0[:, 0::2], W1[:, 0::2]], axis=1)
    wb = jnp.concatenate([W0[:, 1::2], W1[:, 1::2]], axis=1)
    ba = jnp.concatenate([b0[0::2], b1[0::2]])[None, :]
    bb = jnp.concatenate([b0[1::2], b1[1::2]])[None, :]
    awa = alpha_w[:, 0::2]
    awb = alpha_w[:, 1::2]

    grid = (pl.cdiv(n, BM),)
    return pl.pallas_call(
        _fused_body,
        grid=grid,
        in_specs=[
            pl.BlockSpec((BM, n), lambda i: (i, 0)),
            pl.BlockSpec((n, F), lambda i: (0, 0)),
            pl.BlockSpec((F, C), lambda i: (0, 0)),
            pl.BlockSpec((1, C), lambda i: (0, 0)),
            pl.BlockSpec((F, C), lambda i: (0, 0)),
            pl.BlockSpec((1, C), lambda i: (0, 0)),
            pl.BlockSpec((1, C), lambda i: (0, 0)),
            pl.BlockSpec((1, C), lambda i: (0, 0)),
        ],
        out_specs=pl.BlockSpec((BM, C), lambda i: (i, 0)),
        out_shape=jax.ShapeDtypeStruct((n, C), jnp.float32),
        compiler_params=pltpu.CompilerParams(
            dimension_semantics=("arbitrary",),
        ),
    )(fltr, features, wa, ba, wb, bb, awa, awb)


# speedup vs baseline: 1.0046x; 1.0046x over previous
import functools

import jax
import jax.numpy as jnp
from jax.experimental import pallas as pl
from jax.experimental.pallas import tpu as pltpu

F = 128
C = 128
BM = 320


def _fused_body(fltr_ref, feat_ref, wa_ref, ba_ref, wb_ref, bb_ref,
                awa_ref, awb_ref, out_ref):
    h = jnp.dot(fltr_ref[...], feat_ref[...],
                preferred_element_type=jnp.float32)
    a = jnp.maximum(
        jnp.dot(h, wa_ref[...], preferred_element_type=jnp.float32)
        + ba_ref[...], 0.0)
    b = jnp.maximum(
        jnp.dot(h, wb_ref[...], preferred_element_type=jnp.float32)
        + bb_ref[...], 0.0)
    mx = jnp.maximum(a, b)
    av = 0.5 * (a + b)
    gate = jax.nn.sigmoid(
        jnp.sum(a * awa_ref[...] + b * awb_ref[...], axis=1, keepdims=True))
    pooled = gate * mx + (1.0 - gate) * av
    y = jnp.where(pooled > 0, pooled, jnp.exp(pooled) - 1.0) + pooled
    out_ref[...] = jnp.where(y > 0, y, jnp.exp(y) - 1.0)


@functools.partial(jax.jit, static_argnames=())
def kernel(features, fltr, W0, b0, W1, b1, alpha_w):
    n = fltr.shape[0]
    wa = jnp.concatenate([W0[:, 0::2], W1[:, 0::2]], axis=1)
    wb = jnp.concatenate([W0[:, 1::2], W1[:, 1::2]], axis=1)
    ba = jnp.concatenate([b0[0::2], b1[0::2]])[None, :]
    bb = jnp.concatenate([b0[1::2], b1[1::2]])[None, :]
    awa = alpha_w[:, 0::2]
    awb = alpha_w[:, 1::2]

    grid = (pl.cdiv(n, BM),)
    return pl.pallas_call(
        _fused_body,
        grid=grid,
        in_specs=[
            pl.BlockSpec((BM, n), lambda i: (i, 0)),
            pl.BlockSpec((n, F), lambda i: (0, 0)),
            pl.BlockSpec((F, C), lambda i: (0, 0)),
            pl.BlockSpec((1, C), lambda i: (0, 0)),
            pl.BlockSpec((F, C), lambda i: (0, 0)),
            pl.BlockSpec((1, C), lambda i: (0, 0)),
            pl.BlockSpec((1, C), lambda i: (0, 0)),
            pl.BlockSpec((1, C), lambda i: (0, 0)),
        ],
        out_specs=pl.BlockSpec((BM, C), lambda i: (i, 0)),
        out_shape=jax.ShapeDtypeStruct((n, C), jnp.float32),
        compiler_params=pltpu.CompilerParams(
            dimension_semantics=("parallel",),
        ),
    )(fltr, features, wa, ba, wb, bb, awa, awb)
